# Initial kernel scaffold; baseline (speedup 1.0000x reference)
import jax
import jax.numpy as jnp
from jax.experimental import pallas as pl


def kernel(X_train, X_test, y_train):
    raise NotImplementedError("write your pallas kernel here")


# speedup vs baseline: 1.3949x; 1.3949x over previous
import functools

import jax
import jax.numpy as jnp
from jax.experimental import pallas as pl
from jax.experimental.pallas import tpu as pltpu

_NUM_CLASSES = 100
_TOPK = 5
_BLK = 2048
_Q = 1024
_IMAX = 2**31 - 1


def _knn_kernel(n_train, n_steps, xt_ref, xq_ref, y_ref, out_ref,
                vals_ref, idx_ref, lab_ref):
    i = pl.program_id(0)
    blk = xt_ref.shape[0]
    q = xq_ref.shape[0]

    @pl.when(i == 0)
    def _init():
        vals_ref[...] = jnp.full_like(vals_ref[...], jnp.inf)
        idx_ref[...] = jnp.full_like(idx_ref[...], _IMAX)
        lab_ref[...] = jnp.full_like(lab_ref[...], _IMAX)

    xq = xq_ref[...]
    xt = xt_ref[...]
    q_sq = jnp.sum(xq * xq, axis=1, keepdims=True)
    k_sq = jnp.sum(xt * xt, axis=1)[None, :]
    dot = jax.lax.dot_general(xq, xt, (((1,), (1,)), ((), ())),
                              preferred_element_type=jnp.float32)
    d2 = q_sq + k_sq - 2.0 * dot
    dist = jnp.sqrt(jnp.maximum(d2, 0.0))

    gidx = i * blk + jax.lax.broadcasted_iota(jnp.int32, (1, blk), 1)
    valid = gidx < n_train
    dist = jnp.where(valid, dist, jnp.inf)
    gidx_b = jnp.where(valid, jnp.broadcast_to(gidx, (q, blk)), _IMAX)
    labs_b = jnp.broadcast_to(y_ref[0], (q, blk))

    vals = jnp.concatenate([dist, vals_ref[...]], axis=1)
    idxs = jnp.concatenate([gidx_b, idx_ref[...]], axis=1)
    labs = jnp.concatenate([labs_b, lab_ref[...]], axis=1)

    new_v, new_i, new_l = [], [], []
    for _ in range(_TOPK):
        m = jnp.min(vals, axis=1, keepdims=True)
        tie = vals == m
        isel = jnp.min(jnp.where(tie, idxs, _IMAX), axis=1, keepdims=True)
        sel = tie & (idxs == isel)
        lsel = jnp.min(jnp.where(sel, labs, _IMAX), axis=1, keepdims=True)
        vals = jnp.where(sel, jnp.inf, vals)
        new_v.append(m)
        new_i.append(isel)
        new_l.append(lsel)

    vals_ref[...] = jnp.concatenate(new_v, axis=1)
    idx_ref[...] = jnp.concatenate(new_i, axis=1)
    lab_ref[...] = jnp.concatenate(new_l, axis=1)

    @pl.when(i == n_steps - 1)
    def _finish():
        counts = []
        for a in range(_TOPK):
            c = jnp.zeros((q, 1), jnp.int32)
            for b in range(_TOPK):
                c = c + (new_l[a] == new_l[b]).astype(jnp.int32)
            counts.append(c)
        best_s = counts[0] * (_NUM_CLASSES * 10) - new_l[0]
        best_l = new_l[0]
        for a in range(1, _TOPK):
            s = counts[a] * (_NUM_CLASSES * 10) - new_l[a]
            upd = s > best_s
            best_s = jnp.where(upd, s, best_s)
            best_l = jnp.where(upd, new_l[a], best_l)
        out_ref[...] = best_l


@functools.partial(jax.jit, static_argnums=())
def kernel(X_train, X_test, y_train):
    n_train = X_train.shape[0]
    n_steps = (n_train + _BLK - 1) // _BLK
    n_pad = n_steps * _BLK
    xt = jnp.pad(X_train, ((0, n_pad - n_train), (0, 0)))
    y3 = jnp.pad(y_train, (0, n_pad - n_train)).reshape(n_steps, 1, _BLK)

    out = pl.pallas_call(
        functools.partial(_knn_kernel, n_train, n_steps),
        grid=(n_steps,),
        in_specs=[
            pl.BlockSpec((_BLK, 32), lambda i: (i, 0)),
            pl.BlockSpec((_Q, 32), lambda i: (0, 0)),
            pl.BlockSpec((1, 1, _BLK), lambda i: (i, 0, 0)),
        ],
        out_specs=pl.BlockSpec((_Q, 1), lambda i: (0, 0)),
        out_shape=jax.ShapeDtypeStruct((_Q, 1), jnp.int32),
        scratch_shapes=[
            pltpu.VMEM((_Q, _TOPK), jnp.float32),
            pltpu.VMEM((_Q, _TOPK), jnp.int32),
            pltpu.VMEM((_Q, _TOPK), jnp.int32),
        ],
        compiler_params=pltpu.CompilerParams(
            dimension_semantics=("arbitrary",)),
    )(xt, X_test, y3)
    return out[:, 0]


# speedup vs baseline: 2.0844x; 1.4943x over previous
import functools

import jax
import jax.numpy as jnp
from jax import lax
from jax.experimental import pallas as pl
from jax.experimental.pallas import tpu as pltpu
from jax.experimental.pallas import tpu_sc as plsc

_NUM_CLASSES = 100
_TOPK = 5
_BLK = 4096
_Q = 1024
_CHUNK = 128
_CPB = _BLK // _CHUNK
_GRP = 32
_IMAX = 2**31 - 1


def _chunkmin_kernel(n_train, xt_ref, xq_ref, out_ref):
    i = pl.program_id(0)
    blk = xt_ref.shape[0]
    q = xq_ref.shape[0]
    xq = xq_ref[...]
    xt = xt_ref[...]
    q_sq = jnp.sum(xq * xq, axis=1, keepdims=True)
    k_sq = jnp.sum(xt * xt, axis=1)[None, :]
    dot = lax.dot_general(xq, xt, (((1,), (1,)), ((), ())),
                          preferred_element_type=jnp.float32)
    d2 = q_sq + k_sq - 2.0 * dot
    dist = jnp.sqrt(jnp.maximum(d2, 0.0))
    gidx = i * blk + lax.broadcasted_iota(jnp.int32, (1, blk), 1)
    dist = jnp.where(gidx < n_train, dist, jnp.inf)
    out_ref[...] = jnp.min(dist.reshape(q, _CPB, _CHUNK), axis=2)[None]


def _chunksel_kernel(cmin_ref, out_ref):
    vals = cmin_ref[...]
    q, nc = vals.shape
    ciota = jnp.broadcast_to(
        lax.broadcasted_iota(jnp.int32, (1, nc), 1), (q, nc))
    picks = []
    for _ in range(_TOPK):
        m = jnp.min(vals, axis=1, keepdims=True)
        csel = jnp.min(jnp.where(vals == m, ciota, _IMAX),
                       axis=1, keepdims=True)
        vals = jnp.where(ciota == csel, jnp.inf, vals)
        picks.append(csel)
    out_ref[...] = jnp.concatenate(picks, axis=1)


def _make_gather(n_sel):
    info = plsc.get_sparse_core_info()
    nc, ns = info.num_cores, info.num_subcores
    nw = nc * ns
    per_w = n_sel // nw
    batch = 16
    n_batch = per_w // batch
    mesh = plsc.VectorSubcoreMesh(core_axis_name="c", subcore_axis_name="s")

    @functools.partial(
        pl.kernel, mesh=mesh,
        out_type=[
            jax.ShapeDtypeStruct((n_sel, _CHUNK * 32), jnp.float32),
            jax.ShapeDtypeStruct((n_sel, _CHUNK), jnp.int32),
        ],
        scratch_types=[
            pltpu.VMEM((batch,), jnp.int32),
            pltpu.VMEM((batch, _CHUNK * 32), jnp.float32),
            pltpu.VMEM((batch, _CHUNK), jnp.int32),
            pltpu.SemaphoreType.DMA,
        ],
    )
    def gather(xtab_hbm, ytab_hbm, idx_hbm, outx_hbm, outy_hbm,
               idx_v, xbuf, ybuf, sem):
        wid = lax.axis_index("s") * nc + lax.axis_index("c")
        base = wid * per_w
        for b in range(n_batch):
            off = base + b * batch
            pltpu.sync_copy(idx_hbm.at[pl.ds(off, batch)], idx_v)
            pltpu.async_copy(xtab_hbm.at[idx_v], xbuf, sem).wait()
            pltpu.sync_copy(xbuf, outx_hbm.at[pl.ds(off, batch)])
            pltpu.async_copy(ytab_hbm.at[idx_v], ybuf, sem).wait()
            pltpu.sync_copy(ybuf, outy_hbm.at[pl.ds(off, batch)])

    return gather


def _rerank_kernel(n_train, xq_ref, xg_ref, lab_ref, cid_ref, out_ref):
    g = xq_ref.shape[0]
    ncand = _TOPK * _CHUNK
    xq = xq_ref[...]
    xg = xg_ref[...]
    q_sq = jnp.sum(xq * xq, axis=1, keepdims=True)
    k_sq = jnp.sum(xg * xg, axis=1)[None, :]
    dot = lax.dot_general(xq, xg, (((1,), (1,)), ((), ())),
                          preferred_element_type=jnp.float32)
    d2 = q_sq + k_sq - 2.0 * dot
    dist_all = jnp.sqrt(jnp.maximum(d2, 0.0))
    d3 = dist_all.reshape(g, g, ncand)
    own = (lax.broadcasted_iota(jnp.int32, (g, g, 1), 0)
           == lax.broadcasted_iota(jnp.int32, (g, g, 1), 1))
    dist = jnp.min(jnp.where(own, d3, jnp.inf), axis=1)

    cid = cid_ref[0]
    lane = lax.broadcasted_iota(jnp.int32, (1, 1, _CHUNK), 2)
    idxs = (cid[:, :, None] * _CHUNK + lane).reshape(g, ncand)
    labs = lab_ref[...].reshape(g, ncand)
    dist = jnp.where(idxs < n_train, dist, jnp.inf)

    vals = dist
    sel_l = []
    for _ in range(_TOPK):
        m = jnp.min(vals, axis=1, keepdims=True)
        isel = jnp.min(jnp.where(vals == m, idxs, _IMAX),
                       axis=1, keepdims=True)
        sel = idxs == isel
        sel_l.append(jnp.min(jnp.where(sel, labs, _IMAX),
                             axis=1, keepdims=True))
        vals = jnp.where(sel, jnp.inf, vals)

    counts = []
    for a in range(_TOPK):
        c = jnp.zeros((g, 1), jnp.int32)
        for b in range(_TOPK):
            c = c + (sel_l[a] == sel_l[b]).astype(jnp.int32)
        counts.append(c)
    best_s = counts[0] * (_NUM_CLASSES * 10) - sel_l[0]
    best_l = sel_l[0]
    for a in range(1, _TOPK):
        s = counts[a] * (_NUM_CLASSES * 10) - sel_l[a]
        upd = s > best_s
        best_s = jnp.where(upd, s, best_s)
        best_l = jnp.where(upd, sel_l[a], best_l)
    out_ref[...] = best_l


@jax.jit
def kernel(X_train, X_test, y_train):
    n_train = X_train.shape[0]
    n_steps = (n_train + _BLK - 1) // _BLK
    n_pad = n_steps * _BLK
    n_chunks = n_pad // _CHUNK
    xt = jnp.pad(X_train, ((0, n_pad - n_train), (0, 0)))
    yt = jnp.pad(y_train, (0, n_pad - n_train))

    cmin3 = pl.pallas_call(
        functools.partial(_chunkmin_kernel, n_train),
        grid=(n_steps,),
        in_specs=[
            pl.BlockSpec((_BLK, 32), lambda i: (i, 0)),
            pl.BlockSpec((_Q, 32), lambda i: (0, 0)),
        ],
        out_specs=pl.BlockSpec((1, _Q, _CPB), lambda i: (i, 0, 0)),
        out_shape=jax.ShapeDtypeStruct((n_steps, _Q, _CPB), jnp.float32),
        compiler_params=pltpu.CompilerParams(
            dimension_semantics=("arbitrary",)),
    )(xt, X_test)
    cmin = cmin3.transpose(1, 0, 2).reshape(_Q, n_chunks)

    qtile = 256
    cids = pl.pallas_call(
        _chunksel_kernel,
        grid=(_Q // qtile,),
        in_specs=[pl.BlockSpec((qtile, n_chunks), lambda i: (i, 0))],
        out_specs=pl.BlockSpec((qtile, _TOPK), lambda i: (i, 0)),
        out_shape=jax.ShapeDtypeStruct((_Q, _TOPK), jnp.int32),
    )(cmin)

    n_sel = _Q * _TOPK
    idx_flat = cids.reshape(n_sel)
    xtab = xt.reshape(n_chunks, _CHUNK * 32)
    ytab = yt.reshape(n_chunks, _CHUNK)
    xg, yg = _make_gather(n_sel)(xtab, ytab, idx_flat)

    xg2 = xg.reshape(n_sel * _CHUNK, 32)
    cids3 = cids.reshape(_Q // _GRP, _GRP, _TOPK)
    out = pl.pallas_call(
        functools.partial(_rerank_kernel, n_train),
        grid=(_Q // _GRP,),
        in_specs=[
            pl.BlockSpec((_GRP, 32), lambda i: (i, 0)),
            pl.BlockSpec((_GRP * _TOPK * _CHUNK, 32), lambda i: (i, 0)),
            pl.BlockSpec((_GRP * _TOPK, _CHUNK), lambda i: (i, 0)),
            pl.BlockSpec((1, _GRP, _TOPK), lambda i: (i, 0, 0)),
        ],
        out_specs=pl.BlockSpec((_GRP, 1), lambda i: (i, 0)),
        out_shape=jax.ShapeDtypeStruct((_Q, 1), jnp.int32),
    )(X_test, xg2, yg, cids3)
    return out[:, 0]
